# baseline (speedup 1.0000x reference)
import functools

import jax
import jax.numpy as jnp
from jax import lax
from jax.experimental import pallas as pl
from jax.experimental.pallas import tpu as pltpu
from jax.experimental.pallas import tpu_sc as plsc

A = 10000
B = 320000
NEI = 32
H = 128
AF = 128
BF = 144

NW = 32
A_PAD = 10240
B_PAD = 327680
L = 16

_MESH = dict(core_axis_name="c", subcore_axis_name="s")


def _tc_matmul(x, w, bias=None, relu_in=False, relu_out=False, bm=2048):
    M, K = x.shape
    N = w.shape[1]
    assert M % bm == 0

    def body(*refs):
        if bias is None:
            x_ref, w_ref, o_ref = refs
            b_ref = None
        else:
            x_ref, w_ref, b_ref, o_ref = refs
        xv = x_ref[...]
        if relu_in:
            xv = jnp.maximum(xv, 0.0)
        acc = jnp.dot(xv, w_ref[...], preferred_element_type=jnp.float32)
        if b_ref is not None:
            acc = acc + b_ref[...]
        if relu_out:
            acc = jnp.maximum(acc, 0.0)
        o_ref[...] = acc

    in_specs = [
        pl.BlockSpec((bm, K), lambda i: (i, 0)),
        pl.BlockSpec((K, N), lambda i: (0, 0)),
    ]
    args = [x, w]
    if bias is not None:
        in_specs.append(pl.BlockSpec((bm, N), lambda i: (i, 0)))
        args.append(bias)
    return pl.pallas_call(
        body,
        grid=(M // bm,),
        in_specs=in_specs,
        out_specs=pl.BlockSpec((bm, N), lambda i: (i, 0)),
        out_shape=jax.ShapeDtypeStruct((M, N), jnp.float32),
    )(*args)


def _sc_gather_sum(table, idx_flat, bias):
    T, D = table.shape
    G = D // L
    apw = A_PAD // NW
    CH = 4
    n_ch = apw // CH

    scratch = [
        pltpu.VMEM((CH * NEI,), jnp.int32),
        pltpu.VMEM((CH * NEI, D), jnp.float32),
        pltpu.VMEM((CH, D), jnp.float32),
        pltpu.SemaphoreType.DMA,
    ]
    out_type = jax.ShapeDtypeStruct((A_PAD, D), jnp.float32)

    def body(table_h, idx_h, *rest):
        if bias is None:
            (out_h, idx_v, rows_v, acc_v, sem) = rest
            bias_h = None
        else:
            (bias_h, out_h, idx_v, rows_v, acc_v, sem) = rest
        wid = lax.axis_index("s") * 2 + lax.axis_index("c")
        base = wid * apw

        def chunk(i, carry):
            a0 = base + i * CH
            pltpu.sync_copy(idx_h.at[pl.ds(a0 * NEI, CH * NEI)], idx_v)
            pltpu.async_copy(table_h.at[idx_v], rows_v, sem).wait()
            if bias_h is not None:
                pltpu.sync_copy(bias_h.at[pl.ds(a0, CH)], acc_v)
            for a in range(CH):
                for g in range(G):
                    col = pl.ds(g * L, L)
                    if bias_h is not None:
                        v = acc_v[a, col] + rows_v[a * NEI, col]
                    else:
                        v = rows_v[a * NEI, col]
                    for r in range(1, NEI):
                        v = v + rows_v[a * NEI + r, col]
                    acc_v[a, col] = v
            pltpu.sync_copy(acc_v, out_h.at[pl.ds(a0, CH)])
            return carry

        lax.fori_loop(0, n_ch, chunk, 0)

    mesh = plsc.VectorSubcoreMesh(**_MESH)
    fn = functools.partial(pl.kernel, mesh=mesh, out_type=out_type,
                           scratch_types=scratch)(body)
    if bias is None:
        return fn(table, idx_flat)
    return fn(table, idx_flat, bias)


def _sc_gather_i32(table, idx):
    n = idx.shape[0]
    npw = n // NW
    CH = 128
    n_ch = npw // CH

    @functools.partial(
        pl.kernel,
        mesh=plsc.VectorSubcoreMesh(**_MESH),
        out_type=jax.ShapeDtypeStruct((n,), jnp.int32),
        scratch_types=[
            pltpu.VMEM((CH,), jnp.int32),
            pltpu.VMEM((CH,), jnp.int32),
            pltpu.SemaphoreType.DMA,
        ],
    )
    def k(table_h, idx_h, out_h, idx_v, val_v, sem):
        wid = lax.axis_index("s") * 2 + lax.axis_index("c")
        base = wid * npw

        def chunk(i, carry):
            off = base + i * CH
            pltpu.sync_copy(idx_h.at[pl.ds(off, CH)], idx_v)
            pltpu.async_copy(table_h.at[idx_v], val_v, sem).wait()
            pltpu.sync_copy(val_v, out_h.at[pl.ds(off, CH)])
            return carry

        lax.fori_loop(0, n_ch, chunk, 0)

    return k(table, idx)


def _sc_bond_update(inp, Ptab, Mtab, Ftab, b2a_idx, brev_idx, rb_idx):
    bpw = B_PAD // NW
    CH = 128
    n_ch = bpw // CH

    @functools.partial(
        pl.kernel,
        mesh=plsc.VectorSubcoreMesh(**_MESH),
        out_type=jax.ShapeDtypeStruct((B_PAD, H), jnp.float32),
        scratch_types=[
            pltpu.VMEM((CH,), jnp.int32),
            pltpu.VMEM((CH,), jnp.int32),
            pltpu.VMEM((CH,), jnp.int32),
            pltpu.VMEM((CH, H), jnp.float32),
            pltpu.VMEM((CH, H), jnp.float32),
            pltpu.VMEM((CH, H), jnp.float32),
            pltpu.VMEM((CH, H), jnp.float32),
            pltpu.SemaphoreType.DMA,
        ],
    )
    def k(inp_h, P_h, M_h, F_h, ia_h, ir_h, ib_h, out_h,
          i1, i2, i3, binp, bP, bM, bF, sem):
        wid = lax.axis_index("s") * 2 + lax.axis_index("c")
        base = wid * bpw

        def chunk(c, carry):
            off = base + c * CH
            c0 = pltpu.async_copy(inp_h.at[pl.ds(off, CH)], binp, sem)
            pltpu.sync_copy(ia_h.at[pl.ds(off, CH)], i1)
            pltpu.sync_copy(ir_h.at[pl.ds(off, CH)], i2)
            pltpu.sync_copy(ib_h.at[pl.ds(off, CH)], i3)
            c1 = pltpu.async_copy(P_h.at[i1], bP, sem)
            c2 = pltpu.async_copy(M_h.at[i2], bM, sem)
            c3 = pltpu.async_copy(F_h.at[i3], bF, sem)
            c0.wait()
            c1.wait()
            c2.wait()
            c3.wait()

            def row(r, rc):
                for g in range(H // L):
                    col = pl.ds(g * L, L)
                    v = binp[r, col] + bP[r, col] - bM[r, col] - bF[r, col]
                    binp[r, col] = jnp.maximum(v, 0.0)
                return rc

            lax.fori_loop(0, CH, row, 0)
            pltpu.sync_copy(binp, out_h.at[pl.ds(off, CH)])
            return carry

        lax.fori_loop(0, n_ch, chunk, 0)

    return k(inp, Ptab, Mtab, Ftab, b2a_idx, brev_idx, rb_idx)


def kernel(f_atoms, f_bonds, a2b, b2a, b2revb, a2a,
           W_i_atom, W_h_atom, W_i_bond, W_h_bond):
    i32 = jnp.int32
    a2b = a2b.astype(i32)
    b2a = b2a.astype(i32)
    b2revb = b2revb.astype(i32)
    a2a = a2a.astype(i32)

    fA = jnp.pad(f_atoms, ((0, A_PAD - A), (0, 0)))
    fB = jnp.pad(f_bonds, ((0, B_PAD - B), (0, 0)))
    a2b_f = jnp.pad(a2b, ((0, A_PAD - A), (0, 0))).reshape(-1)
    a2a_f = jnp.pad(a2a, ((0, A_PAD - A), (0, 0))).reshape(-1)
    b2a_p = jnp.pad(b2a, (0, B_PAD - B))
    brev_p = jnp.pad(b2revb, (0, B_PAD - B))

    Wh1a, Wh2a = W_h_atom[:H], W_h_atom[H:]
    Wh1b, Wh2b = W_h_bond[:H], W_h_bond[H:]

    inp_b = _tc_matmul(fB, W_i_bond)
    inp_a = _tc_matmul(fA, W_i_atom, bm=2048)
    FA2 = _tc_matmul(fA, Wh2b, bm=2048)

    FB2 = _tc_matmul(fB, Wh2a)
    ba = _sc_gather_sum(FB2, a2b_f, inp_a)
    S_a = _sc_gather_sum(fA, a2a_f, None)
    rb = _sc_gather_i32(b2a_p, brev_p)

    Cb = _tc_matmul(S_a, Wh2b, bm=2048)

    x = inp_b
    for _ in range(2):
        M1 = _tc_matmul(x, Wh1b, relu_in=True)
        P = _sc_gather_sum(M1, a2b_f, Cb)
        x = _sc_bond_update(inp_b, P, M1, FA2, b2a_p, brev_p, rb)
    edge_output = x[:B]

    m = _tc_matmul(fA, W_i_atom, relu_out=True, bm=2048)
    for _ in range(2):
        T = _sc_gather_sum(m, a2a_f, None)
        m = _tc_matmul(T, Wh1a, bias=ba, relu_out=True, bm=2048)
    atom_output = m[:A]

    return (atom_output, edge_output)


# speedup vs baseline: 1.2582x; 1.2582x over previous
import functools

import jax
import jax.numpy as jnp
from jax import lax
from jax.experimental import pallas as pl
from jax.experimental.pallas import tpu as pltpu
from jax.experimental.pallas import tpu_sc as plsc

A = 10000
B = 320000
NEI = 32
H = 128
AF = 128
BF = 144

NW = 32
A_PAD = 10240
B_PAD = 327680
L = 16

_MESH = dict(core_axis_name="c", subcore_axis_name="s")


def _tc_matmul(x, w, bias=None, relu_in=False, relu_out=False, bm=2048):
    M, K = x.shape
    N = w.shape[1]
    assert M % bm == 0

    def body(*refs):
        if bias is None:
            x_ref, w_ref, o_ref = refs
            b_ref = None
        else:
            x_ref, w_ref, b_ref, o_ref = refs
        xv = x_ref[...]
        if relu_in:
            xv = jnp.maximum(xv, 0.0)
        acc = jnp.dot(xv, w_ref[...], preferred_element_type=jnp.float32)
        if b_ref is not None:
            acc = acc + b_ref[...]
        if relu_out:
            acc = jnp.maximum(acc, 0.0)
        o_ref[...] = acc

    in_specs = [
        pl.BlockSpec((bm, K), lambda i: (i, 0)),
        pl.BlockSpec((K, N), lambda i: (0, 0)),
    ]
    args = [x, w]
    if bias is not None:
        in_specs.append(pl.BlockSpec((bm, N), lambda i: (i, 0)))
        args.append(bias)
    return pl.pallas_call(
        body,
        grid=(M // bm,),
        in_specs=in_specs,
        out_specs=pl.BlockSpec((bm, N), lambda i: (i, 0)),
        out_shape=jax.ShapeDtypeStruct((M, N), jnp.float32),
    )(*args)


def _sc_gather_sum(table, idx2d, bias):
    T, D = table.shape
    G = D // L
    apw = A_PAD // NW
    CH = 4
    n_ch = apw // CH
    CHN = CH * NEI

    scratch = [
        pltpu.VMEM((n_ch, CHN), jnp.int32),
        pltpu.VMEM((2, CHN, D), jnp.float32),
        pltpu.VMEM((apw, D), jnp.float32),
        pltpu.SemaphoreType.DMA,
        pltpu.SemaphoreType.DMA,
    ]
    out_type = jax.ShapeDtypeStruct((A_PAD, D), jnp.float32)

    def body(table_h, idx_h, *rest):
        if bias is None:
            (out_h, idx_v, rows_v, out_s, sem0, sem1) = rest
            bias_h = None
        else:
            (bias_h, out_h, idx_v, rows_v, out_s, sem0, sem1) = rest
        sems = (sem0, sem1)
        wid = lax.axis_index("s") * 2 + lax.axis_index("c")
        base = wid * apw

        pltpu.sync_copy(idx_h.at[pl.ds(wid * n_ch, n_ch)], idx_v)
        if bias_h is not None:
            pltpu.sync_copy(bias_h.at[pl.ds(base, apw)], out_s)

        def fire(i, p):
            pltpu.make_async_copy(
                table_h.at[idx_v.at[i]], rows_v.at[p], sems[p]).start()

        def drain(p):
            pltpu.make_async_copy(
                table_h.at[idx_v.at[0]], rows_v.at[p], sems[p]).wait()

        fire(0, 0)

        def pair(t, carry):
            for b in (0, 1):
                i = 2 * t + b
                p, q = b, 1 - b

                @pl.when(i + 1 < n_ch)
                def _():
                    fire(i + 1, q)

                drain(p)
                rows = rows_v.at[p]
                for a in range(CH):
                    row = i * CH + a
                    for g in range(G):
                        col = pl.ds(g * L, L)
                        if bias_h is not None:
                            v = out_s[row, col] + rows[a * NEI, col]
                        else:
                            v = rows[a * NEI, col]
                        for r in range(1, NEI):
                            v = v + rows[a * NEI + r, col]
                        out_s[row, col] = v
            return carry

        lax.fori_loop(0, n_ch // 2, pair, 0)
        pltpu.sync_copy(out_s, out_h.at[pl.ds(base, apw)])

    mesh = plsc.VectorSubcoreMesh(**_MESH)
    fn = functools.partial(pl.kernel, mesh=mesh, out_type=out_type,
                           scratch_types=scratch)(body)
    if bias is None:
        return fn(table, idx2d)
    return fn(table, idx2d, bias)


def _sc_gather_i32(table, idx2d):
    n = B_PAD
    npw = n // NW
    CH = 128
    n_ch = npw // CH

    @functools.partial(
        pl.kernel,
        mesh=plsc.VectorSubcoreMesh(**_MESH),
        out_type=jax.ShapeDtypeStruct((NW * n_ch, CH), jnp.int32),
        scratch_types=[
            pltpu.VMEM((n_ch, CH), jnp.int32),
            pltpu.VMEM((n_ch, CH), jnp.int32),
            pltpu.SemaphoreType.DMA,
            pltpu.SemaphoreType.DMA,
        ],
    )
    def k(table_h, idx_h, out_h, idx_v, val_v, sem0, sem1):
        sems = (sem0, sem1)
        wid = lax.axis_index("s") * 2 + lax.axis_index("c")
        base = wid * npw

        pltpu.sync_copy(idx_h.at[pl.ds(wid * n_ch, n_ch)], idx_v)

        def fire(i, p):
            pltpu.make_async_copy(
                table_h.at[idx_v.at[i]], val_v.at[i], sems[p]).start()

        def drain(p):
            pltpu.make_async_copy(
                table_h.at[idx_v.at[0]], val_v.at[0], sems[p]).wait()

        fire(0, 0)

        def pair(t, carry):
            for b in (0, 1):
                i = 2 * t + b
                p, q = b, 1 - b

                @pl.when(i + 1 < n_ch)
                def _():
                    fire(i + 1, q)

                drain(p)
            return carry

        lax.fori_loop(0, n_ch // 2, pair, 0)
        pltpu.sync_copy(val_v, out_h.at[pl.ds(wid * n_ch, n_ch)])

    return k(table, idx2d)


def _sc_combine(lin, gathers, relu):
    NG = len(gathers)
    bpw = B_PAD // NW
    CH = 128
    n_ch = bpw // CH

    scratch = []
    for _ in range(NG):
        scratch.append(pltpu.VMEM((n_ch, CH), jnp.int32))
    scratch.append(pltpu.VMEM((2, CH, H), jnp.float32))
    for _ in range(NG):
        scratch.append(pltpu.VMEM((2, CH, H), jnp.float32))
    scratch += [pltpu.SemaphoreType.DMA, pltpu.SemaphoreType.DMA,
                pltpu.SemaphoreType.DMA, pltpu.SemaphoreType.DMA]

    @functools.partial(
        pl.kernel,
        mesh=plsc.VectorSubcoreMesh(**_MESH),
        out_type=jax.ShapeDtypeStruct((B_PAD, H), jnp.float32),
        scratch_types=scratch,
    )
    def k(*refs):
        lin_h = refs[0]
        tab_h = refs[1:1 + NG]
        idx_h = refs[1 + NG:1 + 2 * NG]
        r = 2 + 2 * NG
        out_h = refs[r - 1]
        slab_v = refs[r:r + NG]
        lin_v = refs[r + NG]
        g_v = refs[r + NG + 1:r + 2 * NG + 1]
        isem0, isem1, osem0, osem1 = refs[r + 2 * NG + 1:]
        isems = (isem0, isem1)
        osems = (osem0, osem1)

        wid = lax.axis_index("s") * 2 + lax.axis_index("c")
        base = wid * bpw
        for kk in range(NG):
            pltpu.sync_copy(idx_h[kk].at[pl.ds(wid * n_ch, n_ch)], slab_v[kk])

        def fire(i, p):
            pltpu.make_async_copy(
                lin_h.at[pl.ds(base + i * CH, CH)], lin_v.at[p],
                isems[p]).start()
            for kk in range(NG):
                pltpu.make_async_copy(
                    tab_h[kk].at[slab_v[kk].at[i]], g_v[kk].at[p],
                    isems[p]).start()

        def drain_in(p):
            for _ in range(1 + NG):
                pltpu.make_async_copy(
                    lin_h.at[pl.ds(base, CH)], lin_v.at[p], isems[p]).wait()

        def fire_out(j, q):
            pltpu.make_async_copy(
                lin_v.at[q], out_h.at[pl.ds(base + j * CH, CH)],
                osems[q]).start()

        def drain_out(p):
            pltpu.make_async_copy(
                lin_v.at[p], out_h.at[pl.ds(base, CH)], osems[p]).wait()

        fire(0, 0)

        def pair(t, carry):
            for b in (0, 1):
                i = 2 * t + b
                p, q = b, 1 - b

                @pl.when(i >= 2)
                def _():
                    drain_out(p)

                @pl.when(jnp.logical_and(i >= 1, i < n_ch))
                def _():
                    fire(i, p)

                @pl.when(jnp.logical_and(i >= 1, i <= n_ch))
                def _():
                    j = i - 1
                    drain_in(q)
                    lv = lin_v.at[q]

                    def row(rr, rc):
                        for g in range(H // L):
                            col = pl.ds(g * L, L)
                            v = lv[rr, col]
                            for kk in range(NG):
                                gv = g_v[kk].at[q]
                                if gathers[kk][2] > 0:
                                    v = v + gv[rr, col]
                                else:
                                    v = v - gv[rr, col]
                            if relu:
                                v = jnp.maximum(v, 0.0)
                            lv[rr, col] = v
                        return rc

                    lax.fori_loop(0, CH, row, 0)
                    fire_out(j, q)
            return carry

        lax.fori_loop(0, (n_ch + 2) // 2, pair, 0)

    tabs = [g[0] for g in gathers]
    idxs = [g[1] for g in gathers]
    return k(lin, *tabs, *idxs)


def kernel(f_atoms, f_bonds, a2b, b2a, b2revb, a2a,
           W_i_atom, W_h_atom, W_i_bond, W_h_bond):
    i32 = jnp.int32
    a2b = a2b.astype(i32)
    b2a = b2a.astype(i32)
    b2revb = b2revb.astype(i32)
    a2a = a2a.astype(i32)

    fA = jnp.pad(f_atoms, ((0, A_PAD - A), (0, 0)))
    fB = jnp.pad(f_bonds, ((0, B_PAD - B), (0, 0)))
    a2b_2d = jnp.pad(a2b, ((0, A_PAD - A), (0, 0))).reshape(-1, 128)
    a2a_2d = jnp.pad(a2a, ((0, A_PAD - A), (0, 0))).reshape(-1, 128)
    b2a_2d = jnp.pad(b2a, (0, B_PAD - B)).reshape(-1, 128)
    brev_2d = jnp.pad(b2revb, (0, B_PAD - B)).reshape(-1, 128)
    b2a_p = jnp.pad(b2a, (0, B_PAD - B))

    Wh1a, Wh2a = W_h_atom[:H], W_h_atom[H:]
    Wh1b, Wh2b = W_h_bond[:H], W_h_bond[H:]

    inp_b = _tc_matmul(fB, W_i_bond)
    inp_a = _tc_matmul(fA, W_i_atom, bm=2048)
    FA2 = _tc_matmul(fA, Wh2b, bm=2048)

    FB2 = _tc_matmul(fB, Wh2a)
    ba = _sc_gather_sum(FB2, a2b_2d, inp_a)
    S_a = _sc_gather_sum(fA, a2a_2d, None)
    rb = _sc_gather_i32(b2a_p, brev_2d)

    Cb = _tc_matmul(S_a, Wh2b, bm=2048)
    E = _sc_combine(inp_b, [(FA2, rb, -1)], relu=False)

    x = inp_b
    for _ in range(2):
        M1 = _tc_matmul(x, Wh1b, relu_in=True)
        P = _sc_gather_sum(M1, a2b_2d, Cb)
        x = _sc_combine(E, [(P, b2a_2d, 1), (M1, brev_2d, -1)], relu=True)
    edge_output = x[:B]

    m = _tc_matmul(fA, W_i_atom, relu_out=True, bm=2048)
    for _ in range(2):
        T = _sc_gather_sum(m, a2a_2d, None)
        m = _tc_matmul(T, Wh1a, bias=ba, relu_out=True, bm=2048)
    atom_output = m[:A]

    return (atom_output, edge_output)


# speedup vs baseline: 1.2858x; 1.0219x over previous
import functools

import jax
import jax.numpy as jnp
from jax import lax
from jax.experimental import pallas as pl
from jax.experimental.pallas import tpu as pltpu
from jax.experimental.pallas import tpu_sc as plsc

A = 10000
B = 320000
NEI = 32
H = 128
AF = 128
BF = 144

NW = 32
A_PAD = 10240
B_PAD = 327680
L = 16

_MESH = dict(core_axis_name="c", subcore_axis_name="s")


def _tc_matmul(x, w, bias=None, relu_in=False, relu_out=False, bm=2048):
    M, K = x.shape
    N = w.shape[1]
    assert M % bm == 0

    def body(*refs):
        if bias is None:
            x_ref, w_ref, o_ref = refs
            b_ref = None
        else:
            x_ref, w_ref, b_ref, o_ref = refs
        xv = x_ref[...]
        if relu_in:
            xv = jnp.maximum(xv, 0.0)
        acc = jnp.dot(xv, w_ref[...], preferred_element_type=jnp.float32)
        if b_ref is not None:
            acc = acc + b_ref[...]
        if relu_out:
            acc = jnp.maximum(acc, 0.0)
        o_ref[...] = acc

    in_specs = [
        pl.BlockSpec((bm, K), lambda i: (i, 0)),
        pl.BlockSpec((K, N), lambda i: (0, 0)),
    ]
    args = [x, w]
    if bias is not None:
        in_specs.append(pl.BlockSpec((bm, N), lambda i: (i, 0)))
        args.append(bias)
    return pl.pallas_call(
        body,
        grid=(M // bm,),
        in_specs=in_specs,
        out_specs=pl.BlockSpec((bm, N), lambda i: (i, 0)),
        out_shape=jax.ShapeDtypeStruct((M, N), jnp.float32),
    )(*args)


def _sc_gather_sum(table, idx2d, bias):
    T, D = table.shape
    G = D // L
    apw = A_PAD // NW
    CH = 4
    n_ch = apw // CH
    CHN = CH * NEI
    DEPTH = 4

    scratch = [
        pltpu.VMEM((n_ch, CHN), jnp.int32),
        pltpu.VMEM((DEPTH, CHN, D), jnp.float32),
        pltpu.VMEM((apw, D), jnp.float32),
    ] + [pltpu.SemaphoreType.DMA] * DEPTH
    out_type = jax.ShapeDtypeStruct((A_PAD, D), jnp.float32)

    def body(table_h, idx_h, *rest):
        if bias is None:
            (out_h, idx_v, rows_v, out_s) = rest[:4]
            sems = rest[4:]
            bias_h = None
        else:
            (bias_h, out_h, idx_v, rows_v, out_s) = rest[:5]
            sems = rest[5:]
        wid = lax.axis_index("s") * 2 + lax.axis_index("c")
        base = wid * apw

        pltpu.sync_copy(idx_h.at[pl.ds(wid * n_ch, n_ch)], idx_v)
        if bias_h is not None:
            pltpu.sync_copy(bias_h.at[pl.ds(base, apw)], out_s)

        def fire(i, p):
            pltpu.make_async_copy(
                table_h.at[idx_v.at[i]], rows_v.at[p], sems[p]).start()

        def drain(p):
            pltpu.make_async_copy(
                table_h.at[idx_v.at[0]], rows_v.at[p], sems[p]).wait()

        for p0 in range(DEPTH - 1):
            fire(p0, p0)

        def grp(t, carry):
            for b in range(DEPTH):
                i = t * DEPTH + b
                p = b

                @pl.when(i + DEPTH - 1 < n_ch)
                def _():
                    fire(i + DEPTH - 1, (p + DEPTH - 1) % DEPTH)

                drain(p)
                rows = rows_v.at[p]

                def atom(a, ac):
                    row = i * CH + a
                    for g in range(G):
                        col = pl.ds(g * L, L)
                        if bias_h is not None:
                            init = out_s[row, col]
                        else:
                            init = jnp.zeros((L,), jnp.float32)

                        def rstep(r, v):
                            rr = a * NEI + r * 4
                            return (v + rows[rr, col] + rows[rr + 1, col]
                                    + rows[rr + 2, col] + rows[rr + 3, col])

                        out_s[row, col] = lax.fori_loop(0, NEI // 4, rstep, init)
                    return ac

                lax.fori_loop(0, CH, atom, 0)
            return carry

        lax.fori_loop(0, n_ch // DEPTH, grp, 0)
        pltpu.sync_copy(out_s, out_h.at[pl.ds(base, apw)])

    mesh = plsc.VectorSubcoreMesh(**_MESH)
    fn = functools.partial(pl.kernel, mesh=mesh, out_type=out_type,
                           scratch_types=scratch)(body)
    if bias is None:
        return fn(table, idx2d)
    return fn(table, idx2d, bias)


def _sc_gather_i32(table, idx2d):
    n = B_PAD
    npw = n // NW
    CH = 128
    n_ch = npw // CH

    @functools.partial(
        pl.kernel,
        mesh=plsc.VectorSubcoreMesh(**_MESH),
        out_type=jax.ShapeDtypeStruct((NW * n_ch, CH), jnp.int32),
        scratch_types=[
            pltpu.VMEM((n_ch, CH), jnp.int32),
            pltpu.VMEM((n_ch, CH), jnp.int32),
            pltpu.SemaphoreType.DMA,
            pltpu.SemaphoreType.DMA,
        ],
    )
    def k(table_h, idx_h, out_h, idx_v, val_v, sem0, sem1):
        sems = (sem0, sem1)
        wid = lax.axis_index("s") * 2 + lax.axis_index("c")
        base = wid * npw

        pltpu.sync_copy(idx_h.at[pl.ds(wid * n_ch, n_ch)], idx_v)

        def fire(i, p):
            pltpu.make_async_copy(
                table_h.at[idx_v.at[i]], val_v.at[i], sems[p]).start()

        def drain(p):
            pltpu.make_async_copy(
                table_h.at[idx_v.at[0]], val_v.at[0], sems[p]).wait()

        fire(0, 0)

        def pair(t, carry):
            for b in (0, 1):
                i = 2 * t + b
                p, q = b, 1 - b

                @pl.when(i + 1 < n_ch)
                def _():
                    fire(i + 1, q)

                drain(p)
            return carry

        lax.fori_loop(0, n_ch // 2, pair, 0)
        pltpu.sync_copy(val_v, out_h.at[pl.ds(wid * n_ch, n_ch)])

    return k(table, idx2d)


def _sc_combine(lin, gathers, relu):
    NG = len(gathers)
    bpw = B_PAD // NW
    CH = 128
    n_ch = bpw // CH

    scratch = []
    for _ in range(NG):
        scratch.append(pltpu.VMEM((n_ch, CH), jnp.int32))
    scratch.append(pltpu.VMEM((2, CH, H), jnp.float32))
    for _ in range(NG):
        scratch.append(pltpu.VMEM((2, CH, H), jnp.float32))
    scratch += [pltpu.SemaphoreType.DMA, pltpu.SemaphoreType.DMA,
                pltpu.SemaphoreType.DMA, pltpu.SemaphoreType.DMA]

    @functools.partial(
        pl.kernel,
        mesh=plsc.VectorSubcoreMesh(**_MESH),
        out_type=jax.ShapeDtypeStruct((B_PAD, H), jnp.float32),
        scratch_types=scratch,
    )
    def k(*refs):
        lin_h = refs[0]
        tab_h = refs[1:1 + NG]
        idx_h = refs[1 + NG:1 + 2 * NG]
        r = 2 + 2 * NG
        out_h = refs[r - 1]
        slab_v = refs[r:r + NG]
        lin_v = refs[r + NG]
        g_v = refs[r + NG + 1:r + 2 * NG + 1]
        isem0, isem1, osem0, osem1 = refs[r + 2 * NG + 1:]
        isems = (isem0, isem1)
        osems = (osem0, osem1)

        wid = lax.axis_index("s") * 2 + lax.axis_index("c")
        base = wid * bpw
        for kk in range(NG):
            pltpu.sync_copy(idx_h[kk].at[pl.ds(wid * n_ch, n_ch)], slab_v[kk])

        def fire(i, p):
            pltpu.make_async_copy(
                lin_h.at[pl.ds(base + i * CH, CH)], lin_v.at[p],
                isems[p]).start()
            for kk in range(NG):
                pltpu.make_async_copy(
                    tab_h[kk].at[slab_v[kk].at[i]], g_v[kk].at[p],
                    isems[p]).start()

        def drain_in(p):
            for _ in range(1 + NG):
                pltpu.make_async_copy(
                    lin_h.at[pl.ds(base, CH)], lin_v.at[p], isems[p]).wait()

        def fire_out(j, q):
            pltpu.make_async_copy(
                lin_v.at[q], out_h.at[pl.ds(base + j * CH, CH)],
                osems[q]).start()

        def drain_out(p):
            pltpu.make_async_copy(
                lin_v.at[p], out_h.at[pl.ds(base, CH)], osems[p]).wait()

        fire(0, 0)

        def pair(t, carry):
            for b in (0, 1):
                i = 2 * t + b
                p, q = b, 1 - b

                @pl.when(i >= 2)
                def _():
                    drain_out(p)

                @pl.when(jnp.logical_and(i >= 1, i < n_ch))
                def _():
                    fire(i, p)

                @pl.when(jnp.logical_and(i >= 1, i <= n_ch))
                def _():
                    j = i - 1
                    drain_in(q)
                    lv = lin_v.at[q]

                    def row(rr, rc):
                        for g in range(H // L):
                            col = pl.ds(g * L, L)
                            v = lv[rr, col]
                            for kk in range(NG):
                                gv = g_v[kk].at[q]
                                if gathers[kk][2] > 0:
                                    v = v + gv[rr, col]
                                else:
                                    v = v - gv[rr, col]
                            if relu:
                                v = jnp.maximum(v, 0.0)
                            lv[rr, col] = v
                        return rc

                    lax.fori_loop(0, CH, row, 0)
                    fire_out(j, q)
            return carry

        lax.fori_loop(0, (n_ch + 2) // 2, pair, 0)

    tabs = [g[0] for g in gathers]
    idxs = [g[1] for g in gathers]
    return k(lin, *tabs, *idxs)


def kernel(f_atoms, f_bonds, a2b, b2a, b2revb, a2a,
           W_i_atom, W_h_atom, W_i_bond, W_h_bond):
    i32 = jnp.int32
    a2b = a2b.astype(i32)
    b2a = b2a.astype(i32)
    b2revb = b2revb.astype(i32)
    a2a = a2a.astype(i32)

    fA = jnp.pad(f_atoms, ((0, A_PAD - A), (0, 0)))
    fB = jnp.pad(f_bonds, ((0, B_PAD - B), (0, 0)))
    a2b_2d = jnp.pad(a2b, ((0, A_PAD - A), (0, 0))).reshape(-1, 128)
    a2a_2d = jnp.pad(a2a, ((0, A_PAD - A), (0, 0))).reshape(-1, 128)
    b2a_2d = jnp.pad(b2a, (0, B_PAD - B)).reshape(-1, 128)
    brev_2d = jnp.pad(b2revb, (0, B_PAD - B)).reshape(-1, 128)
    b2a_p = jnp.pad(b2a, (0, B_PAD - B))

    Wh1a, Wh2a = W_h_atom[:H], W_h_atom[H:]
    Wh1b, Wh2b = W_h_bond[:H], W_h_bond[H:]

    inp_b = _tc_matmul(fB, W_i_bond)
    inp_a = _tc_matmul(fA, W_i_atom, bm=2048)
    FA2 = _tc_matmul(fA, Wh2b, bm=2048)

    FB2 = _tc_matmul(fB, Wh2a)
    ba = _sc_gather_sum(FB2, a2b_2d, inp_a)
    S_a = _sc_gather_sum(fA, a2a_2d, None)
    rb = _sc_gather_i32(b2a_p, brev_2d)

    Cb = _tc_matmul(S_a, Wh2b, bm=2048)
    E = _sc_combine(inp_b, [(FA2, rb, -1)], relu=False)

    x = inp_b
    for _ in range(2):
        M1 = _tc_matmul(x, Wh1b, relu_in=True)
        P = _sc_gather_sum(M1, a2b_2d, Cb)
        x = _sc_combine(E, [(P, b2a_2d, 1), (M1, brev_2d, -1)], relu=True)
    edge_output = x[:B]

    m = _tc_matmul(fA, W_i_atom, relu_out=True, bm=2048)
    for _ in range(2):
        T = _sc_gather_sum(m, a2a_2d, None)
        m = _tc_matmul(T, Wh1a, bias=ba, relu_out=True, bm=2048)
    atom_output = m[:A]

    return (atom_output, edge_output)


# speedup vs baseline: 1.2912x; 1.0042x over previous
import functools

import jax
import jax.numpy as jnp
from jax import lax
from jax.experimental import pallas as pl
from jax.experimental.pallas import tpu as pltpu
from jax.experimental.pallas import tpu_sc as plsc

A = 10000
B = 320000
NEI = 32
H = 128
AF = 128
BF = 144

NW = 32
A_PAD = 10240
B_PAD = 327680
L = 16

_MESH = dict(core_axis_name="c", subcore_axis_name="s")


def _tc_matmul(x, w, bias=None, relu_in=False, relu_out=False, bm=2048):
    M, K = x.shape
    N = w.shape[1]
    assert M % bm == 0

    def body(*refs):
        if bias is None:
            x_ref, w_ref, o_ref = refs
            b_ref = None
        else:
            x_ref, w_ref, b_ref, o_ref = refs
        xv = x_ref[...]
        if relu_in:
            xv = jnp.maximum(xv, 0.0)
        acc = jnp.dot(xv, w_ref[...], preferred_element_type=jnp.float32)
        if b_ref is not None:
            acc = acc + b_ref[...]
        if relu_out:
            acc = jnp.maximum(acc, 0.0)
        o_ref[...] = acc

    in_specs = [
        pl.BlockSpec((bm, K), lambda i: (i, 0)),
        pl.BlockSpec((K, N), lambda i: (0, 0)),
    ]
    args = [x, w]
    if bias is not None:
        in_specs.append(pl.BlockSpec((bm, N), lambda i: (i, 0)))
        args.append(bias)
    return pl.pallas_call(
        body,
        grid=(M // bm,),
        in_specs=in_specs,
        out_specs=pl.BlockSpec((bm, N), lambda i: (i, 0)),
        out_shape=jax.ShapeDtypeStruct((M, N), jnp.float32),
    )(*args)


def _tc_dual(x1, w1, x2, w2, relu1=False, relu_in2=False, copy2=False,
             bm=2048):
    M = x1.shape[0]
    K1 = x1.shape[1]
    K2 = x2.shape[1]

    def body(x1_ref, w1_ref, x2_ref, w2_ref, o_ref):
        a = jnp.dot(x1_ref[...], w1_ref[...],
                    preferred_element_type=jnp.float32)
        if relu1:
            a = jnp.maximum(a, 0.0)
        if copy2:
            b = x2_ref[...]
        else:
            xv = x2_ref[...]
            if relu_in2:
                xv = jnp.maximum(xv, 0.0)
            b = jnp.dot(xv, w2_ref[...], preferred_element_type=jnp.float32)
        o_ref[...] = jnp.concatenate([a, b], axis=1)

    return pl.pallas_call(
        body,
        grid=(M // bm,),
        in_specs=[
            pl.BlockSpec((bm, K1), lambda i: (i, 0)),
            pl.BlockSpec((K1, H), lambda i: (0, 0)),
            pl.BlockSpec((bm, K2), lambda i: (i, 0)),
            pl.BlockSpec((K2, H), lambda i: (0, 0)),
        ],
        out_specs=pl.BlockSpec((bm, 2 * H), lambda i: (i, 0)),
        out_shape=jax.ShapeDtypeStruct((M, 2 * H), jnp.float32),
    )(x1, w1, x2, w2)


def _sc_gather_sum(table, idx_flat, bias, stage=False):
    T, D = table.shape
    G = D // L
    apw = A_PAD // NW
    if D <= 128:
        CH, DEPTH = 4, 4
    else:
        CH, DEPTH = 2, 2
    n_ch = apw // CH
    CHN = CH * NEI
    idx2d = idx_flat.reshape(NW * n_ch, CHN)

    scratch = [
        pltpu.VMEM((n_ch, CHN), jnp.int32),
        pltpu.VMEM((DEPTH, CHN, D), jnp.float32),
        pltpu.VMEM((apw, D), jnp.float32),
    ] + [pltpu.SemaphoreType.DMA] * DEPTH
    if stage:
        scratch.append(pltpu.VMEM_SHARED((T, D), jnp.float32))
    out_type = jax.ShapeDtypeStruct((A_PAD, D), jnp.float32)

    def body(table_h, idx_h, *rest):
        if bias is None:
            (out_h, idx_v, rows_v, out_s) = rest[:4]
            sems = rest[4:]
            bias_h = None
        else:
            (bias_h, out_h, idx_v, rows_v, out_s) = rest[:5]
            sems = rest[5:]
        if stage:
            shared = sems[-1]
            sems = sems[:-1]
        wid = lax.axis_index("s") * 2 + lax.axis_index("c")
        base = wid * apw

        if stage:
            @pl.when(lax.axis_index("s") == 0)
            def _():
                pltpu.sync_copy(table_h, shared)
            src = shared
        else:
            src = table_h

        pltpu.sync_copy(idx_h.at[pl.ds(wid * n_ch, n_ch)], idx_v)
        if bias_h is not None:
            pltpu.sync_copy(bias_h.at[pl.ds(base, apw)], out_s)
        if stage:
            plsc.subcore_barrier()

        def fire(i, p):
            pltpu.make_async_copy(
                src.at[idx_v.at[i]], rows_v.at[p], sems[p]).start()

        def drain(p):
            pltpu.make_async_copy(
                src.at[idx_v.at[0]], rows_v.at[p], sems[p]).wait()

        for p0 in range(DEPTH - 1):
            fire(p0, p0)

        def grp(t, carry):
            for b in range(DEPTH):
                i = t * DEPTH + b
                p = b

                @pl.when(i + DEPTH - 1 < n_ch)
                def _():
                    fire(i + DEPTH - 1, (p + DEPTH - 1) % DEPTH)

                drain(p)
                rows = rows_v.at[p]

                def atom(a, ac):
                    row = i * CH + a
                    for g in range(G):
                        col = pl.ds(g * L, L)
                        if bias_h is not None:
                            init = out_s[row, col]
                        else:
                            init = jnp.zeros((L,), jnp.float32)

                        def rstep(r, v):
                            rr = a * NEI + r * 4
                            return (v + rows[rr, col] + rows[rr + 1, col]
                                    + rows[rr + 2, col] + rows[rr + 3, col])

                        out_s[row, col] = lax.fori_loop(0, NEI // 4, rstep, init)
                    return ac

                lax.fori_loop(0, CH, atom, 0)
            return carry

        lax.fori_loop(0, n_ch // DEPTH, grp, 0)
        pltpu.sync_copy(out_s, out_h.at[pl.ds(base, apw)])

    mesh = plsc.VectorSubcoreMesh(**_MESH)
    fn = functools.partial(pl.kernel, mesh=mesh, out_type=out_type,
                           scratch_types=scratch)(body)
    if bias is None:
        return fn(table, idx2d)
    return fn(table, idx2d, bias)


def _sc_gather_i32(table, idx2d):
    n = B_PAD
    npw = n // NW
    CH = 128
    n_ch = npw // CH

    @functools.partial(
        pl.kernel,
        mesh=plsc.VectorSubcoreMesh(**_MESH),
        out_type=jax.ShapeDtypeStruct((NW * n_ch, CH), jnp.int32),
        scratch_types=[
            pltpu.VMEM((n_ch, CH), jnp.int32),
            pltpu.VMEM((n_ch, CH), jnp.int32),
            pltpu.SemaphoreType.DMA,
            pltpu.SemaphoreType.DMA,
            pltpu.VMEM_SHARED((B_PAD,), jnp.int32),
        ],
    )
    def k(table_h, idx_h, out_h, idx_v, val_v, sem0, sem1, shared):
        sems = (sem0, sem1)
        wid = lax.axis_index("s") * 2 + lax.axis_index("c")
        base = wid * npw

        @pl.when(lax.axis_index("s") == 0)
        def _():
            pltpu.sync_copy(table_h, shared)

        pltpu.sync_copy(idx_h.at[pl.ds(wid * n_ch, n_ch)], idx_v)
        plsc.subcore_barrier()

        def fire(i, p):
            pltpu.make_async_copy(
                shared.at[idx_v.at[i]], val_v.at[i], sems[p]).start()

        def drain(p):
            pltpu.make_async_copy(
                shared.at[idx_v.at[0]], val_v.at[0], sems[p]).wait()

        fire(0, 0)

        def pair(t, carry):
            for b in (0, 1):
                i = 2 * t + b
                p, q = b, 1 - b

                @pl.when(i + 1 < n_ch)
                def _():
                    fire(i + 1, q)

                drain(p)
            return carry

        lax.fori_loop(0, n_ch // 2, pair, 0)
        pltpu.sync_copy(val_v, out_h.at[pl.ds(wid * n_ch, n_ch)])

    return k(table, idx2d)


def _sc_combine(lin, gathers, relu):
    NG = len(gathers)
    bpw = B_PAD // NW
    CH = 128
    n_ch = bpw // CH
    staged = [bool(g[3]) for g in gathers]

    scratch = []
    for _ in range(NG):
        scratch.append(pltpu.VMEM((n_ch, CH), jnp.int32))
    scratch.append(pltpu.VMEM((2, CH, H), jnp.float32))
    for _ in range(NG):
        scratch.append(pltpu.VMEM((2, CH, H), jnp.float32))
    scratch += [pltpu.SemaphoreType.DMA, pltpu.SemaphoreType.DMA,
                pltpu.SemaphoreType.DMA, pltpu.SemaphoreType.DMA]
    for kk in range(NG):
        if staged[kk]:
            scratch.append(
                pltpu.VMEM_SHARED(gathers[kk][0].shape, jnp.float32))

    @functools.partial(
        pl.kernel,
        mesh=plsc.VectorSubcoreMesh(**_MESH),
        out_type=jax.ShapeDtypeStruct((B_PAD, H), jnp.float32),
        scratch_types=scratch,
    )
    def k(*refs):
        lin_h = refs[0]
        tab_h = refs[1:1 + NG]
        idx_h = refs[1 + NG:1 + 2 * NG]
        r = 2 + 2 * NG
        out_h = refs[r - 1]
        slab_v = refs[r:r + NG]
        lin_v = refs[r + NG]
        g_v = refs[r + NG + 1:r + 2 * NG + 1]
        sems = refs[r + 2 * NG + 1:]
        isems = (sems[0], sems[1])
        osems = (sems[2], sems[3])
        shared = list(sems[4:])

        wid = lax.axis_index("s") * 2 + lax.axis_index("c")
        base = wid * bpw

        tab_src = []
        for kk in range(NG):
            if staged[kk]:
                sh = shared.pop(0)

                @pl.when(lax.axis_index("s") == 0)
                def _(tsrc=tab_h[kk], tdst=sh):
                    pltpu.sync_copy(tsrc, tdst)
                tab_src.append(sh)
            else:
                tab_src.append(tab_h[kk])
        for kk in range(NG):
            pltpu.sync_copy(idx_h[kk].at[pl.ds(wid * n_ch, n_ch)], slab_v[kk])
        if any(staged):
            plsc.subcore_barrier()

        def fire(i, p):
            pltpu.make_async_copy(
                lin_h.at[pl.ds(base + i * CH, CH)], lin_v.at[p],
                isems[p]).start()
            for kk in range(NG):
                pltpu.make_async_copy(
                    tab_src[kk].at[slab_v[kk].at[i]], g_v[kk].at[p],
                    isems[p]).start()

        def drain_in(p):
            for _ in range(1 + NG):
                pltpu.make_async_copy(
                    lin_h.at[pl.ds(base, CH)], lin_v.at[p], isems[p]).wait()

        def fire_out(j, q):
            pltpu.make_async_copy(
                lin_v.at[q], out_h.at[pl.ds(base + j * CH, CH)],
                osems[q]).start()

        def drain_out(p):
            pltpu.make_async_copy(
                lin_v.at[p], out_h.at[pl.ds(base, CH)], osems[p]).wait()

        fire(0, 0)

        def pair(t, carry):
            for b in (0, 1):
                i = 2 * t + b
                p, q = b, 1 - b

                @pl.when(i >= 2)
                def _():
                    drain_out(p)

                @pl.when(jnp.logical_and(i >= 1, i < n_ch))
                def _():
                    fire(i, p)

                @pl.when(jnp.logical_and(i >= 1, i <= n_ch))
                def _():
                    j = i - 1
                    drain_in(q)
                    lv = lin_v.at[q]

                    def row(rr, rc):
                        for g in range(H // L):
                            col = pl.ds(g * L, L)
                            v = lv[rr, col]
                            for kk in range(NG):
                                gv = g_v[kk].at[q]
                                if gathers[kk][2] > 0:
                                    v = v + gv[rr, col]
                                else:
                                    v = v - gv[rr, col]
                            if relu:
                                v = jnp.maximum(v, 0.0)
                            lv[rr, col] = v
                        return rc

                    lax.fori_loop(0, CH, row, 0)
                    fire_out(j, q)
            return carry

        lax.fori_loop(0, (n_ch + 2) // 2, pair, 0)

    tabs = [g[0] for g in gathers]
    idxs = [g[1] for g in gathers]
    return k(lin, *tabs, *idxs)


def kernel(f_atoms, f_bonds, a2b, b2a, b2revb, a2a,
           W_i_atom, W_h_atom, W_i_bond, W_h_bond):
    i32 = jnp.int32
    a2b = a2b.astype(i32)
    b2a = b2a.astype(i32)
    b2revb = b2revb.astype(i32)
    a2a = a2a.astype(i32)

    fA = jnp.pad(f_atoms, ((0, A_PAD - A), (0, 0)))
    fB = jnp.pad(f_bonds, ((0, B_PAD - B), (0, 0)))
    a2b_f = jnp.pad(a2b, ((0, A_PAD - A), (0, 0))).reshape(-1)
    a2a_f = jnp.pad(a2a, ((0, A_PAD - A), (0, 0))).reshape(-1)
    b2a_2d = jnp.pad(b2a, (0, B_PAD - B)).reshape(-1, 128)
    brev_2d = jnp.pad(b2revb, (0, B_PAD - B)).reshape(-1, 128)
    b2a_p = jnp.pad(b2a, (0, B_PAD - B))

    Wh1a, Wh2a = W_h_atom[:H], W_h_atom[H:]
    Wh1b, Wh2b = W_h_bond[:H], W_h_bond[H:]

    inp_b = _tc_matmul(fB, W_i_bond)
    inp_a = _tc_matmul(fA, W_i_atom, bm=2048)
    FA2 = _tc_matmul(fA, Wh2b, bm=2048)

    FB2 = _tc_matmul(fB, Wh2a)
    ba = _sc_gather_sum(FB2, a2b_f, inp_a)
    S_a = _sc_gather_sum(fA, a2a_f, None)
    rb = _sc_gather_i32(b2a_p, brev_2d)

    Cb = _tc_matmul(S_a, Wh2b, bm=2048)
    E = _sc_combine(inp_b, [(FA2, rb, -1, False)], relu=False)

    x = inp_b
    for _ in range(2):
        M1 = _tc_matmul(x, Wh1b, relu_in=True)
        P = _sc_gather_sum(M1, a2b_f, Cb)
        x = _sc_combine(E, [(P, b2a_2d, 1, False), (M1, brev_2d, -1, False)],
                        relu=True)
    edge_output = x[:B]

    m = _tc_matmul(fA, W_i_atom, relu_out=True, bm=2048)
    for _ in range(2):
        T = _sc_gather_sum(m, a2a_f, None)
        m = _tc_matmul(T, Wh1a, bias=ba, relu_out=True, bm=2048)
    atom_output = m[:A]

    return (atom_output, edge_output)
